# baseline (speedup 1.0000x reference)
import dataclasses
import functools

import jax
import jax.numpy as jnp
from jax import lax
from jax.experimental import pallas as pl
from jax.experimental.pallas import tpu as pltpu
from jax.experimental.pallas import tpu_sc as plsc

T, H, E, K, F, I = 2048, 2048, 8, 2, 1664, 3328
BM = 256
PMAX = T * K + E * BM
NBLK = PMAX // BM
TBLK = T // BM
IC = I // 2
NIC = 2
GW = 16
META_NBT = 32
NEG = -1e30


def _sc_compiler_params():
    cp = pltpu.CompilerParams()
    if "needs_layout_passes" in pltpu.CompilerParams.__dataclass_fields__:
        cp = dataclasses.replace(cp, needs_layout_passes=False)
    return cp


def _router_body(x_ref, rwt_ref, idx_ref, w_ref, cnt_ref, xc_ref):
    xb = x_ref[...]
    logits = jnp.dot(xb, rwt_ref[...], preferred_element_type=jnp.float32)
    eio = lax.broadcasted_iota(jnp.int32, (BM, E), 1)
    m1 = jnp.max(logits, axis=1, keepdims=True)
    am1 = jnp.min(jnp.where(logits == m1, eio, E), axis=1, keepdims=True)
    l2 = jnp.where(eio == am1, NEG, logits)
    m2 = jnp.max(l2, axis=1, keepdims=True)
    am2 = jnp.min(jnp.where(l2 == m2, eio, E), axis=1, keepdims=True)
    ew = jnp.exp(m2 - m1)
    w2v = ew / (1.0 + ew)
    w1v = 1.0 - w2v
    xc_ref[...] = xb.astype(jnp.bfloat16)
    idx_ref[0] = jnp.concatenate([am1, am2], axis=1)
    w_ref[0] = jnp.concatenate([w1v, w2v], axis=1)
    oh = (eio == am1).astype(jnp.int32) + (eio == am2).astype(jnp.int32)
    rio = lax.broadcasted_iota(jnp.int32, (BM, E), 0)
    h0 = jnp.sum(jnp.where(rio < BM // 2, oh, 0), axis=0, keepdims=True)
    h1 = jnp.sum(jnp.where(rio >= BM // 2, oh, 0), axis=0, keepdims=True)
    z = jnp.zeros((1, E), jnp.int32)
    cnt_ref[0] = jnp.concatenate(
        [jnp.concatenate([h0, z], axis=1),
         jnp.concatenate([h1, z], axis=1)], axis=0)


def _router(x2d, rwt):
    return pl.pallas_call(
        _router_body,
        grid=(TBLK,),
        in_specs=[
            pl.BlockSpec((BM, H), lambda t: (t, 0)),
            pl.BlockSpec((H, E), lambda t: (0, 0)),
        ],
        out_specs=[
            pl.BlockSpec((1, BM, K), lambda t: (t, 0, 0)),
            pl.BlockSpec((1, BM, K), lambda t: (t, 0, 0)),
            pl.BlockSpec((1, 2, 16), lambda t: (t, 0, 0)),
            pl.BlockSpec((BM, H), lambda t: (t, 0)),
        ],
        out_shape=[
            jax.ShapeDtypeStruct((TBLK, BM, K), jnp.int32),
            jax.ShapeDtypeStruct((TBLK, BM, K), jnp.float32),
            jax.ShapeDtypeStruct((TBLK, 2, 16), jnp.int32),
            jax.ShapeDtypeStruct((T, H), jnp.bfloat16),
        ],
    )(x2d, rwt)


def _sort_sc(idx16, cnt16):
    mesh = plsc.VectorSubcoreMesh(core_axis_name="c", subcore_axis_name="s")

    @functools.partial(
        pl.kernel,
        out_type=[
            jax.ShapeDtypeStruct((16, 256), jnp.int32),
            jax.ShapeDtypeStruct((PMAX,), jnp.int32),
            jax.ShapeDtypeStruct((64,), jnp.int32),
        ],
        mesh=mesh,
        scratch_types=[
            pltpu.VMEM((256,), jnp.int32),
            pltpu.VMEM((256,), jnp.int32),
            pltpu.VMEM((2, 128), jnp.int32),
            pltpu.VMEM((2, 128), jnp.int32),
            pltpu.VMEM((16, 16), jnp.int32),
            pltpu.VMEM((64,), jnp.int32),
            pltpu.SemaphoreType.DMA,
        ],
        compiler_params=_sc_compiler_params(),
    )
    def k(idx_hbm, cnt_hbm, pos_hbm, st_hbm, meta_hbm, ev, posl, posx, tvals,
          allc, metal, sem):
        cid = lax.axis_index("c")
        sid = lax.axis_index("s")
        lanes = lax.broadcasted_iota(jnp.int32, (16,), 0)

        @pl.when(cid == 0)
        def _():
            pltpu.sync_copy(idx_hbm.at[sid], ev)
            pltpu.sync_copy(cnt_hbm, allc)

            tot = jnp.zeros((16,), jnp.int32)
            bef = jnp.zeros((16,), jnp.int32)
            for w in range(16):
                row = allc[w]
                tot = tot + row
                bef = bef + row * (w < sid).astype(jnp.int32)
            padded = ((tot + (BM - 1)) >> 8) << 8
            css = plsc.cumsum(padded)
            start = css - padded
            basev = start + bef

            for v in range(16):
                evv = ev[pl.ds(v * 16, 16)]
                rank = jnp.zeros((16,), jnp.int32)
                hv = jnp.zeros((16,), jnp.int32)
                for e in range(E):
                    m = evv == e
                    cs = plsc.cumsum(m.astype(jnp.int32))
                    rank = rank + jnp.where(m, cs - 1, 0)
                    pc = plsc.all_reduce_population_count(m)
                    hv = hv + jnp.where(lanes == e, pc, 0)
                posv = basev.at[evv].get(mode="promise_in_bounds") + rank
                basev = basev + hv
                posl[pl.ds(v * 16, 16)] = posv
                posx[v // 8, pl.ds((v % 8) * 16, 16)] = posv
                jv = sid * 256 + v * 16 + lanes
                tv = ((jv >> 9) << 8) | ((jv >> 1) & 255)
                tvals[v // 8, pl.ds((v % 8) * 16, 16)] = tv

            pltpu.sync_copy(posl, pos_hbm.at[sid])
            for ch in range(2):
                pltpu.sync_copy(tvals.at[ch], st_hbm.at[posx.at[ch]])

            @pl.when(sid == 0)
            def _():
                nbt = jnp.sum(jnp.where(lanes == E - 1, css, 0)) >> 8
                for r in range(2):
                    bpos = (r * 16 + lanes) << 8
                    acc = jnp.zeros((16,), jnp.int32)
                    for e in range(E):
                        se = jnp.sum(jnp.where(lanes == e, start, 0))
                        acc = acc + (se <= bpos).astype(jnp.int32)
                    metal[pl.ds(r * 16, 16)] = jnp.clip(acc - 1, 0, E - 1)
                nbtv = jnp.zeros((16,), jnp.int32) + nbt
                metal[pl.ds(32, 16)] = nbtv
                metal[pl.ds(48, 16)] = nbtv
                pltpu.sync_copy(metal, meta_hbm)

    return k(idx16, cnt16)


def _gather_rows_sc(src, idx_flat, nrows):
    mesh = plsc.VectorSubcoreMesh(core_axis_name="c", subcore_axis_name="s")
    bf = src.dtype == jnp.bfloat16
    if bf:
        src = lax.bitcast_convert_type(
            src.reshape(src.shape[0], src.shape[1] // 2, 2), jnp.int32)
    nsrc, ncol = src.shape
    bpw = nrows // 32
    ch = 16
    nch = bpw // ch

    @functools.partial(
        pl.kernel,
        out_type=jax.ShapeDtypeStruct((nrows, ncol), src.dtype),
        mesh=mesh,
        scratch_types=[
            pltpu.VMEM((bpw,), jnp.int32),
            pltpu.VMEM((ch, ncol), src.dtype),
            pltpu.VMEM((ch, ncol), src.dtype),
            pltpu.SemaphoreType.DMA,
            pltpu.SemaphoreType.DMA,
        ],
        compiler_params=_sc_compiler_params(),
    )
    def k(src_hbm, i_hbm, o_hbm, idxv, bufa, bufb, sema, semb):
        cid = lax.axis_index("c")
        sid = lax.axis_index("s")
        wid = sid * 2 + cid
        base = wid * bpw
        pltpu.sync_copy(i_hbm.at[pl.ds(base, bpw)], idxv)
        msk = (1 << (nsrc - 1).bit_length()) - 1
        for vv in range(bpw // 16):
            cv = idxv[pl.ds(vv * 16, 16)]
            idxv[pl.ds(vv * 16, 16)] = jnp.minimum(cv & msk, nsrc - 1)
        bufs = [bufa, bufb]
        sems = [sema, semb]
        cps = [None, None]
        cps[0] = pltpu.async_copy(
            src_hbm.at[idxv.at[pl.ds(0, ch)]], bufa, sema)
        for c in range(nch):
            if c + 1 < nch:
                cps[(c + 1) % 2] = pltpu.async_copy(
                    src_hbm.at[idxv.at[pl.ds((c + 1) * ch, ch)]],
                    bufs[(c + 1) % 2], sems[(c + 1) % 2])
            cps[c % 2].wait()
            pltpu.sync_copy(bufs[c % 2], o_hbm.at[pl.ds(base + c * ch, ch)])

    out = k(src, idx_flat)
    if bf:
        out = lax.bitcast_convert_type(out, jnp.bfloat16).reshape(nrows, H)
    return out


def _g1a_body(meta_ref, xs_ref, w1_ref, gsx_ref, w1c, ce):
    i = pl.program_id(0)
    e = meta_ref[i]

    @pl.when(jnp.logical_or(i == 0, ce[0] != e))
    def _():
        w1c[...] = w1_ref[0].astype(jnp.bfloat16)
        ce[0] = e

    @pl.when(i < meta_ref[META_NBT])
    def _():
        gt = jnp.dot(xs_ref[...], w1c[...], preferred_element_type=jnp.float32)
        gsx_ref[...] = (gt * lax.logistic(gt)).astype(jnp.bfloat16)


def _g1a(meta, xs, w1):
    grid_spec = pltpu.PrefetchScalarGridSpec(
        num_scalar_prefetch=1,
        grid=(NBLK,),
        in_specs=[
            pl.BlockSpec((BM, H), lambda i, m: (i, 0)),
            pl.BlockSpec((1, H, F), lambda i, m: (m[i], 0, 0)),
        ],
        out_specs=pl.BlockSpec((BM, F), lambda i, m: (i, 0)),
        scratch_shapes=[
            pltpu.VMEM((H, F), jnp.bfloat16),
            pltpu.SMEM((1,), jnp.int32),
        ],
    )
    return pl.pallas_call(
        _g1a_body,
        grid_spec=grid_spec,
        out_shape=jax.ShapeDtypeStruct((PMAX, F), jnp.bfloat16),
    )(meta, xs, w1)


def _g1b_body(meta_ref, xs_ref, w3_ref, gsx_ref, act_ref, w3c, ce):
    i = pl.program_id(0)
    e = meta_ref[i]

    @pl.when(jnp.logical_or(i == 0, ce[0] != e))
    def _():
        w3c[...] = w3_ref[0].astype(jnp.bfloat16)
        ce[0] = e

    @pl.when(i < meta_ref[META_NBT])
    def _():
        up = jnp.dot(xs_ref[...], w3c[...], preferred_element_type=jnp.float32)
        act_ref[...] = (gsx_ref[...].astype(jnp.float32) * up).astype(
            jnp.bfloat16)


def _g1b(meta, xs, w3, gsx):
    grid_spec = pltpu.PrefetchScalarGridSpec(
        num_scalar_prefetch=1,
        grid=(NBLK,),
        in_specs=[
            pl.BlockSpec((BM, H), lambda i, m: (i, 0)),
            pl.BlockSpec((1, H, F), lambda i, m: (m[i], 0, 0)),
            pl.BlockSpec((BM, F), lambda i, m: (i, 0)),
        ],
        out_specs=pl.BlockSpec((BM, F), lambda i, m: (i, 0)),
        scratch_shapes=[
            pltpu.VMEM((H, F), jnp.bfloat16),
            pltpu.SMEM((1,), jnp.int32),
        ],
    )
    return pl.pallas_call(
        _g1b_body,
        grid_spec=grid_spec,
        out_shape=jax.ShapeDtypeStruct((PMAX, F), jnp.bfloat16),
    )(meta, xs, w3, gsx)


def _g2_body(meta_ref, act_ref, w2_ref, ys_ref, w2c, ce):
    i = pl.program_id(0)
    e = meta_ref[i]

    @pl.when(jnp.logical_or(i == 0, ce[0] != e))
    def _():
        w2c[...] = w2_ref[0].astype(jnp.bfloat16)
        ce[0] = e

    @pl.when(i < meta_ref[META_NBT])
    def _():
        ys_ref[...] = jnp.dot(act_ref[...], w2c[...],
                              preferred_element_type=jnp.float32
                              ).astype(jnp.bfloat16)


def _g2(meta, act, w2):
    grid_spec = pltpu.PrefetchScalarGridSpec(
        num_scalar_prefetch=1,
        grid=(NBLK,),
        in_specs=[
            pl.BlockSpec((BM, F), lambda i, m: (i, 0)),
            pl.BlockSpec((1, F, H), lambda i, m: (m[i], 0, 0)),
        ],
        out_specs=pl.BlockSpec((BM, H), lambda i, m: (i, 0)),
        scratch_shapes=[
            pltpu.VMEM((F, H), jnp.bfloat16),
            pltpu.SMEM((1,), jnp.int32),
        ],
    )
    return pl.pallas_call(
        _g2_body,
        grid_spec=grid_spec,
        out_shape=jax.ShapeDtypeStruct((PMAX, H), jnp.bfloat16),
    )(meta, act, w2)


def _s1a_body(x_ref, sgu_ref, gs_ref, wgc, cc):
    c = pl.program_id(0)
    t = pl.program_id(1)

    @pl.when(jnp.logical_or(jnp.logical_and(c == 0, t == 0), cc[0] != c))
    def _():
        wgc[...] = sgu_ref[...].astype(jnp.bfloat16)
        cc[0] = c

    xb = x_ref[...].astype(jnp.bfloat16)
    g = jnp.dot(xb, wgc[...], preferred_element_type=jnp.float32)
    gs_ref[...] = (g * lax.logistic(g)).astype(jnp.bfloat16)


def _s1a(x2d, sgu):
    return pl.pallas_call(
        _s1a_body,
        grid=(NIC, TBLK),
        in_specs=[
            pl.BlockSpec((BM, H), lambda c, t: (t, 0)),
            pl.BlockSpec((H, IC), lambda c, t: (0, c)),
        ],
        out_specs=pl.BlockSpec((BM, IC), lambda c, t: (t, c)),
        out_shape=jax.ShapeDtypeStruct((T, I), jnp.bfloat16),
        scratch_shapes=[
            pltpu.VMEM((H, IC), jnp.bfloat16),
            pltpu.SMEM((1,), jnp.int32),
        ],
    )(x2d, sgu)


def _s1b_body(x_ref, sgu_ref, gs_ref, act_ref, wuc, cc):
    c = pl.program_id(0)
    t = pl.program_id(1)

    @pl.when(jnp.logical_or(jnp.logical_and(c == 0, t == 0), cc[0] != c))
    def _():
        wuc[...] = sgu_ref[...].astype(jnp.bfloat16)
        cc[0] = c

    xb = x_ref[...].astype(jnp.bfloat16)
    u = jnp.dot(xb, wuc[...], preferred_element_type=jnp.float32)
    act_ref[...] = (gs_ref[...].astype(jnp.float32) * u).astype(jnp.bfloat16)


def _s1b(x2d, sgu, gs):
    return pl.pallas_call(
        _s1b_body,
        grid=(NIC, TBLK),
        in_specs=[
            pl.BlockSpec((BM, H), lambda c, t: (t, 0)),
            pl.BlockSpec((H, IC), lambda c, t: (0, NIC + c)),
            pl.BlockSpec((BM, IC), lambda c, t: (t, c)),
        ],
        out_specs=pl.BlockSpec((BM, IC), lambda c, t: (t, c)),
        out_shape=jax.ShapeDtypeStruct((T, I), jnp.bfloat16),
        scratch_shapes=[
            pltpu.VMEM((H, IC), jnp.bfloat16),
            pltpu.SMEM((1,), jnp.int32),
        ],
    )(x2d, sgu, gs)


def _s2_body(act_ref, sd_ref, g_ref, w_ref, out_ref):
    sh = jnp.dot(act_ref[...], sd_ref[...], preferred_element_type=jnp.float32)
    g0 = g_ref[0, :, 0, :].astype(jnp.float32)
    g1 = g_ref[0, :, 1, :].astype(jnp.float32)
    w0 = w_ref[0, :, 0:1]
    w1 = w_ref[0, :, 1:2]
    out_ref[...] = w0 * g0 + w1 * g1 + sh


def _s2(act, sdb, g4, wv):
    return pl.pallas_call(
        _s2_body,
        grid=(TBLK,),
        in_specs=[
            pl.BlockSpec((BM, I), lambda t: (t, 0)),
            pl.BlockSpec((I, H), lambda t: (0, 0)),
            pl.BlockSpec((1, BM, K, H), lambda t: (t, 0, 0, 0)),
            pl.BlockSpec((1, BM, K), lambda t: (t, 0, 0)),
        ],
        out_specs=pl.BlockSpec((BM, H), lambda t: (t, 0)),
        out_shape=jax.ShapeDtypeStruct((T, H), jnp.float32),
    )(act, sdb, g4, wv)


def kernel(hidden_states, router_weight, w1, w3, w2, shared_gate_up,
           shared_down):
    b, s, h = hidden_states.shape
    x2d = hidden_states.reshape(T, H)
    rwt = router_weight.T

    idx, wv, cnt, xcopy = _router(x2d, rwt)
    idx16 = idx.reshape(16, 256)

    pos, sort_tok, meta = _sort_sc(idx16, cnt.reshape(16, 16))

    xs = _gather_rows_sc(xcopy, sort_tok, PMAX)
    gsx = _g1a(meta, xs, w1)
    act = _g1b(meta, xs, w3, gsx)
    ys = _g2(meta, act, w2)
    g = _gather_rows_sc(ys, pos.reshape(-1), T * K)
    g4 = g.reshape(TBLK, BM, K, H)

    gs = _s1a(x2d, shared_gate_up)
    act_sh = _s1b(x2d, shared_gate_up, gs)
    out = _s2(act_sh, shared_down.astype(jnp.bfloat16), g4, wv)
    return out.reshape(b, s, h)


# speedup vs baseline: 4.6947x; 4.6947x over previous
import dataclasses
import functools

import jax
import jax.numpy as jnp
from jax import lax
from jax.experimental import pallas as pl
from jax.experimental.pallas import tpu as pltpu
from jax.experimental.pallas import tpu_sc as plsc

T, H, E, K, F, I = 2048, 2048, 8, 2, 1664, 3328
BM = 256
PMAX = T * K + E * BM
NBLK = PMAX // BM
TBLK = T // BM
IC = I // 2
NIC = 2
GW = 16
META_NBT = 32
NEG = -1e30


def _sc_compiler_params():
    cp = pltpu.CompilerParams()
    if "needs_layout_passes" in pltpu.CompilerParams.__dataclass_fields__:
        cp = dataclasses.replace(cp, needs_layout_passes=False)
    return cp


def _router_body(x_ref, rwt_ref, idx_ref, w_ref, cnt_ref):
    xb = x_ref[...]
    logits = jnp.dot(xb, rwt_ref[...], preferred_element_type=jnp.float32)
    eio = lax.broadcasted_iota(jnp.int32, (BM, E), 1)
    m1 = jnp.max(logits, axis=1, keepdims=True)
    am1 = jnp.min(jnp.where(logits == m1, eio, E), axis=1, keepdims=True)
    l2 = jnp.where(eio == am1, NEG, logits)
    m2 = jnp.max(l2, axis=1, keepdims=True)
    am2 = jnp.min(jnp.where(l2 == m2, eio, E), axis=1, keepdims=True)
    ew = jnp.exp(m2 - m1)
    w2v = ew / (1.0 + ew)
    w1v = 1.0 - w2v
    idx_ref[0] = jnp.concatenate([am1, am2], axis=1)
    w_ref[0] = jnp.concatenate([w1v, w2v], axis=1)
    oh = (eio == am1).astype(jnp.int32) + (eio == am2).astype(jnp.int32)
    rio = lax.broadcasted_iota(jnp.int32, (BM, E), 0)
    h0 = jnp.sum(jnp.where(rio < BM // 2, oh, 0), axis=0, keepdims=True)
    h1 = jnp.sum(jnp.where(rio >= BM // 2, oh, 0), axis=0, keepdims=True)
    z = jnp.zeros((1, E), jnp.int32)
    cnt_ref[0] = jnp.concatenate(
        [jnp.concatenate([h0, z], axis=1),
         jnp.concatenate([h1, z], axis=1)], axis=0)


def _router(x2d, rwt):
    return pl.pallas_call(
        _router_body,
        grid=(TBLK,),
        in_specs=[
            pl.BlockSpec((BM, H), lambda t: (t, 0)),
            pl.BlockSpec((H, E), lambda t: (0, 0)),
        ],
        out_specs=[
            pl.BlockSpec((1, BM, K), lambda t: (t, 0, 0)),
            pl.BlockSpec((1, BM, K), lambda t: (t, 0, 0)),
            pl.BlockSpec((1, 2, 16), lambda t: (t, 0, 0)),
        ],
        out_shape=[
            jax.ShapeDtypeStruct((TBLK, BM, K), jnp.int32),
            jax.ShapeDtypeStruct((TBLK, BM, K), jnp.float32),
            jax.ShapeDtypeStruct((TBLK, 2, 16), jnp.int32),
        ],
    )(x2d, rwt)


def _sort_sc(idx16, cnt16):
    mesh = plsc.VectorSubcoreMesh(core_axis_name="c", subcore_axis_name="s")

    @functools.partial(
        pl.kernel,
        out_type=[
            jax.ShapeDtypeStruct((16, 256), jnp.int32),
            jax.ShapeDtypeStruct((PMAX,), jnp.int32),
            jax.ShapeDtypeStruct((64,), jnp.int32),
        ],
        mesh=mesh,
        scratch_types=[
            pltpu.VMEM((256,), jnp.int32),
            pltpu.VMEM((256,), jnp.int32),
            pltpu.VMEM((2, 128), jnp.int32),
            pltpu.VMEM((2, 128), jnp.int32),
            pltpu.VMEM((16, 16), jnp.int32),
            pltpu.VMEM((64,), jnp.int32),
            pltpu.SemaphoreType.DMA,
        ],
        compiler_params=_sc_compiler_params(),
    )
    def k(idx_hbm, cnt_hbm, pos_hbm, st_hbm, meta_hbm, ev, posl, posx, tvals,
          allc, metal, sem):
        cid = lax.axis_index("c")
        sid = lax.axis_index("s")
        lanes = lax.broadcasted_iota(jnp.int32, (16,), 0)

        @pl.when(cid == 0)
        def _():
            pltpu.sync_copy(idx_hbm.at[sid], ev)
            pltpu.sync_copy(cnt_hbm, allc)

            tot = jnp.zeros((16,), jnp.int32)
            bef = jnp.zeros((16,), jnp.int32)
            for w in range(16):
                row = allc[w]
                tot = tot + row
                bef = bef + row * (w < sid).astype(jnp.int32)
            padded = ((tot + (BM - 1)) >> 8) << 8
            css = plsc.cumsum(padded)
            start = css - padded
            basev = start + bef

            for v in range(16):
                evv = ev[pl.ds(v * 16, 16)]
                rank = jnp.zeros((16,), jnp.int32)
                hv = jnp.zeros((16,), jnp.int32)
                for e in range(E):
                    m = evv == e
                    cs = plsc.cumsum(m.astype(jnp.int32))
                    rank = rank + jnp.where(m, cs - 1, 0)
                    pc = plsc.all_reduce_population_count(m)
                    hv = hv + jnp.where(lanes == e, pc, 0)
                posv = basev.at[evv].get(mode="promise_in_bounds") + rank
                basev = basev + hv
                posl[pl.ds(v * 16, 16)] = posv
                posx[v // 8, pl.ds((v % 8) * 16, 16)] = posv
                jv = sid * 256 + v * 16 + lanes
                tv = ((jv >> 9) << 8) | ((jv >> 1) & 255)
                tvals[v // 8, pl.ds((v % 8) * 16, 16)] = tv

            pltpu.sync_copy(posl, pos_hbm.at[sid])
            for ch in range(2):
                pltpu.sync_copy(tvals.at[ch], st_hbm.at[posx.at[ch]])

            @pl.when(sid == 0)
            def _():
                nbt = jnp.sum(jnp.where(lanes == E - 1, css, 0)) >> 8
                for r in range(2):
                    bpos = (r * 16 + lanes) << 8
                    acc = jnp.zeros((16,), jnp.int32)
                    for e in range(E):
                        se = jnp.sum(jnp.where(lanes == e, start, 0))
                        acc = acc + (se <= bpos).astype(jnp.int32)
                    metal[pl.ds(r * 16, 16)] = jnp.clip(acc - 1, 0, E - 1)
                nbtv = jnp.zeros((16,), jnp.int32) + nbt
                metal[pl.ds(32, 16)] = nbtv
                metal[pl.ds(48, 16)] = nbtv
                pltpu.sync_copy(metal, meta_hbm)

    return k(idx16, cnt16)


def _gather_rows_sc(src, idx_flat, nrows):
    mesh = plsc.VectorSubcoreMesh(core_axis_name="c", subcore_axis_name="s")
    nsrc = src.shape[0]
    bpw = nrows // 32
    nch = 8
    ch = bpw // nch

    @functools.partial(
        pl.kernel,
        out_type=jax.ShapeDtypeStruct((nrows, H), src.dtype),
        mesh=mesh,
        scratch_types=[
            pltpu.VMEM((bpw,), jnp.int32),
            pltpu.VMEM((ch, H), jnp.float32),
            pltpu.VMEM((ch, H), jnp.float32),
            pltpu.SemaphoreType.DMA,
            pltpu.SemaphoreType.DMA,
        ],
        compiler_params=_sc_compiler_params(),
    )
    def k(src_hbm, i_hbm, o_hbm, idxv, bufa, bufb, sema, semb):
        cid = lax.axis_index("c")
        sid = lax.axis_index("s")
        wid = sid * 2 + cid
        base = wid * bpw
        pltpu.sync_copy(i_hbm.at[pl.ds(base, bpw)], idxv)
        for vv in range(bpw // 16):
            cv = idxv[pl.ds(vv * 16, 16)]
            idxv[pl.ds(vv * 16, 16)] = jnp.clip(cv, 0, nsrc - 1)
        bufs = [bufa, bufb]
        sems = [sema, semb]
        cps = [None, None]
        cps[0] = pltpu.async_copy(
            src_hbm.at[idxv.at[pl.ds(0, ch)]], bufa, sema)
        for c in range(nch):
            if c + 1 < nch:
                cps[(c + 1) % 2] = pltpu.async_copy(
                    src_hbm.at[idxv.at[pl.ds((c + 1) * ch, ch)]],
                    bufs[(c + 1) % 2], sems[(c + 1) % 2])
            cps[c % 2].wait()
            pltpu.sync_copy(bufs[c % 2], o_hbm.at[pl.ds(base + c * ch, ch)])

    return k(src, idx_flat)


def _g1a_body(meta_ref, xs_ref, w1_ref, gsx_ref, w1c, ce):
    i = pl.program_id(0)
    e = meta_ref[i]

    @pl.when(jnp.logical_or(i == 0, ce[0] != e))
    def _():
        w1c[...] = w1_ref[0].astype(jnp.bfloat16)
        ce[0] = e

    @pl.when(i < meta_ref[META_NBT])
    def _():
        xb = xs_ref[...].astype(jnp.bfloat16)
        gt = jnp.dot(xb, w1c[...], preferred_element_type=jnp.float32)
        gsx_ref[...] = (gt * lax.logistic(gt)).astype(jnp.bfloat16)


def _g1a(meta, xs, w1):
    grid_spec = pltpu.PrefetchScalarGridSpec(
        num_scalar_prefetch=1,
        grid=(NBLK,),
        in_specs=[
            pl.BlockSpec((BM, H), lambda i, m: (i, 0)),
            pl.BlockSpec((1, H, F), lambda i, m: (m[i], 0, 0)),
        ],
        out_specs=pl.BlockSpec((BM, F), lambda i, m: (i, 0)),
        scratch_shapes=[
            pltpu.VMEM((H, F), jnp.bfloat16),
            pltpu.SMEM((1,), jnp.int32),
        ],
    )
    return pl.pallas_call(
        _g1a_body,
        grid_spec=grid_spec,
        out_shape=jax.ShapeDtypeStruct((PMAX, F), jnp.bfloat16),
    )(meta, xs, w1)


def _g1b_body(meta_ref, xs_ref, w3_ref, gsx_ref, act_ref, w3c, ce):
    i = pl.program_id(0)
    e = meta_ref[i]

    @pl.when(jnp.logical_or(i == 0, ce[0] != e))
    def _():
        w3c[...] = w3_ref[0].astype(jnp.bfloat16)
        ce[0] = e

    @pl.when(i < meta_ref[META_NBT])
    def _():
        xb = xs_ref[...].astype(jnp.bfloat16)
        up = jnp.dot(xb, w3c[...], preferred_element_type=jnp.float32)
        act_ref[...] = (gsx_ref[...].astype(jnp.float32) * up).astype(
            jnp.bfloat16)


def _g1b(meta, xs, w3, gsx):
    grid_spec = pltpu.PrefetchScalarGridSpec(
        num_scalar_prefetch=1,
        grid=(NBLK,),
        in_specs=[
            pl.BlockSpec((BM, H), lambda i, m: (i, 0)),
            pl.BlockSpec((1, H, F), lambda i, m: (m[i], 0, 0)),
            pl.BlockSpec((BM, F), lambda i, m: (i, 0)),
        ],
        out_specs=pl.BlockSpec((BM, F), lambda i, m: (i, 0)),
        scratch_shapes=[
            pltpu.VMEM((H, F), jnp.bfloat16),
            pltpu.SMEM((1,), jnp.int32),
        ],
    )
    return pl.pallas_call(
        _g1b_body,
        grid_spec=grid_spec,
        out_shape=jax.ShapeDtypeStruct((PMAX, F), jnp.bfloat16),
    )(meta, xs, w3, gsx)


def _g2_body(meta_ref, act_ref, w2_ref, ys_ref, w2c, ce):
    i = pl.program_id(0)
    e = meta_ref[i]

    @pl.when(jnp.logical_or(i == 0, ce[0] != e))
    def _():
        w2c[...] = w2_ref[0].astype(jnp.bfloat16)
        ce[0] = e

    @pl.when(i < meta_ref[META_NBT])
    def _():
        ys_ref[...] = jnp.dot(act_ref[...], w2c[...],
                              preferred_element_type=jnp.float32)


def _g2(meta, act, w2):
    grid_spec = pltpu.PrefetchScalarGridSpec(
        num_scalar_prefetch=1,
        grid=(NBLK,),
        in_specs=[
            pl.BlockSpec((BM, F), lambda i, m: (i, 0)),
            pl.BlockSpec((1, F, H), lambda i, m: (m[i], 0, 0)),
        ],
        out_specs=pl.BlockSpec((BM, H), lambda i, m: (i, 0)),
        scratch_shapes=[
            pltpu.VMEM((F, H), jnp.bfloat16),
            pltpu.SMEM((1,), jnp.int32),
        ],
    )
    return pl.pallas_call(
        _g2_body,
        grid_spec=grid_spec,
        out_shape=jax.ShapeDtypeStruct((PMAX, H), jnp.float32),
    )(meta, act, w2)


def _s1a_body(x_ref, sgu_ref, gs_ref, wgc, cc):
    c = pl.program_id(0)
    t = pl.program_id(1)

    @pl.when(jnp.logical_or(jnp.logical_and(c == 0, t == 0), cc[0] != c))
    def _():
        wgc[...] = sgu_ref[...].astype(jnp.bfloat16)
        cc[0] = c

    xb = x_ref[...].astype(jnp.bfloat16)
    g = jnp.dot(xb, wgc[...], preferred_element_type=jnp.float32)
    gs_ref[...] = (g * lax.logistic(g)).astype(jnp.bfloat16)


def _s1a(x2d, sgu):
    return pl.pallas_call(
        _s1a_body,
        grid=(NIC, TBLK),
        in_specs=[
            pl.BlockSpec((BM, H), lambda c, t: (t, 0)),
            pl.BlockSpec((H, IC), lambda c, t: (0, c)),
        ],
        out_specs=pl.BlockSpec((BM, IC), lambda c, t: (t, c)),
        out_shape=jax.ShapeDtypeStruct((T, I), jnp.bfloat16),
        scratch_shapes=[
            pltpu.VMEM((H, IC), jnp.bfloat16),
            pltpu.SMEM((1,), jnp.int32),
        ],
    )(x2d, sgu)


def _s1b_body(x_ref, sgu_ref, gs_ref, act_ref, wuc, cc):
    c = pl.program_id(0)
    t = pl.program_id(1)

    @pl.when(jnp.logical_or(jnp.logical_and(c == 0, t == 0), cc[0] != c))
    def _():
        wuc[...] = sgu_ref[...].astype(jnp.bfloat16)
        cc[0] = c

    xb = x_ref[...].astype(jnp.bfloat16)
    u = jnp.dot(xb, wuc[...], preferred_element_type=jnp.float32)
    act_ref[...] = (gs_ref[...].astype(jnp.float32) * u).astype(jnp.bfloat16)


def _s1b(x2d, sgu, gs):
    return pl.pallas_call(
        _s1b_body,
        grid=(NIC, TBLK),
        in_specs=[
            pl.BlockSpec((BM, H), lambda c, t: (t, 0)),
            pl.BlockSpec((H, IC), lambda c, t: (0, NIC + c)),
            pl.BlockSpec((BM, IC), lambda c, t: (t, c)),
        ],
        out_specs=pl.BlockSpec((BM, IC), lambda c, t: (t, c)),
        out_shape=jax.ShapeDtypeStruct((T, I), jnp.bfloat16),
        scratch_shapes=[
            pltpu.VMEM((H, IC), jnp.bfloat16),
            pltpu.SMEM((1,), jnp.int32),
        ],
    )(x2d, sgu, gs)


def _s2_body(act_ref, sd_ref, g_ref, w_ref, out_ref):
    sh = jnp.dot(act_ref[...], sd_ref[...], preferred_element_type=jnp.float32)
    g0 = g_ref[0, :, 0, :]
    g1 = g_ref[0, :, 1, :]
    w0 = w_ref[0, :, 0:1]
    w1 = w_ref[0, :, 1:2]
    out_ref[...] = w0 * g0 + w1 * g1 + sh


def _s2(act, sdb, g4, wv):
    return pl.pallas_call(
        _s2_body,
        grid=(TBLK,),
        in_specs=[
            pl.BlockSpec((BM, I), lambda t: (t, 0)),
            pl.BlockSpec((I, H), lambda t: (0, 0)),
            pl.BlockSpec((1, BM, K, H), lambda t: (t, 0, 0, 0)),
            pl.BlockSpec((1, BM, K), lambda t: (t, 0, 0)),
        ],
        out_specs=pl.BlockSpec((BM, H), lambda t: (t, 0)),
        out_shape=jax.ShapeDtypeStruct((T, H), jnp.float32),
    )(act, sdb, g4, wv)


def kernel(hidden_states, router_weight, w1, w3, w2, shared_gate_up,
           shared_down):
    b, s, h = hidden_states.shape
    x2d = hidden_states.reshape(T, H)
    rwt = router_weight.T

    idx, wv, cnt = _router(x2d, rwt)
    idx16 = idx.reshape(16, 256)

    pos, sort_tok, meta = _sort_sc(idx16, cnt.reshape(16, 16))

    xs = _gather_rows_sc(x2d, sort_tok, PMAX)
    gsx = _g1a(meta, xs, w1)
    act = _g1b(meta, xs, w3, gsx)
    ys = _g2(meta, act, w2)
    g = _gather_rows_sc(ys, pos.reshape(-1), T * K)
    g4 = g.reshape(TBLK, BM, K, H)

    gs = _s1a(x2d, shared_gate_up)
    act_sh = _s1b(x2d, shared_gate_up, gs)
    out = _s2(act_sh, shared_down.astype(jnp.bfloat16), g4, wv)
    return out.reshape(b, s, h)


# speedup vs baseline: 4.6960x; 1.0003x over previous
import dataclasses
import functools

import jax
import jax.numpy as jnp
from jax import lax
from jax.experimental import pallas as pl
from jax.experimental.pallas import tpu as pltpu
from jax.experimental.pallas import tpu_sc as plsc

T, H, E, K, F, I = 2048, 2048, 8, 2, 1664, 3328
BM = 256
PMAX = T * K + E * BM
NBLK = PMAX // BM
TBLK = T // BM
IC = I // 2
NIC = 2
GW = 16
META_NBT = 32
NEG = -1e30


def _sc_compiler_params():
    cp = pltpu.CompilerParams()
    if "needs_layout_passes" in pltpu.CompilerParams.__dataclass_fields__:
        cp = dataclasses.replace(cp, needs_layout_passes=False)
    return cp


def _router_body(x_ref, rwt_ref, idx_ref, w_ref, cnt_ref):
    xb = x_ref[...]
    logits = jnp.dot(xb, rwt_ref[...], preferred_element_type=jnp.float32)
    eio = lax.broadcasted_iota(jnp.int32, (BM, E), 1)
    m1 = jnp.max(logits, axis=1, keepdims=True)
    am1 = jnp.min(jnp.where(logits == m1, eio, E), axis=1, keepdims=True)
    l2 = jnp.where(eio == am1, NEG, logits)
    m2 = jnp.max(l2, axis=1, keepdims=True)
    am2 = jnp.min(jnp.where(l2 == m2, eio, E), axis=1, keepdims=True)
    ew = jnp.exp(m2 - m1)
    w2v = ew / (1.0 + ew)
    w1v = 1.0 - w2v
    idx_ref[0] = jnp.concatenate([am1, am2], axis=1)
    w_ref[0] = jnp.concatenate([w1v, w2v], axis=1)
    oh = (eio == am1).astype(jnp.int32) + (eio == am2).astype(jnp.int32)
    rio = lax.broadcasted_iota(jnp.int32, (BM, E), 0)
    h0 = jnp.sum(jnp.where(rio < BM // 2, oh, 0), axis=0, keepdims=True)
    h1 = jnp.sum(jnp.where(rio >= BM // 2, oh, 0), axis=0, keepdims=True)
    z = jnp.zeros((1, E), jnp.int32)
    cnt_ref[0] = jnp.concatenate(
        [jnp.concatenate([h0, z], axis=1),
         jnp.concatenate([h1, z], axis=1)], axis=0)


def _router(x2d, rwt):
    return pl.pallas_call(
        _router_body,
        grid=(TBLK,),
        in_specs=[
            pl.BlockSpec((BM, H), lambda t: (t, 0)),
            pl.BlockSpec((H, E), lambda t: (0, 0)),
        ],
        out_specs=[
            pl.BlockSpec((1, BM, K), lambda t: (t, 0, 0)),
            pl.BlockSpec((1, BM, K), lambda t: (t, 0, 0)),
            pl.BlockSpec((1, 2, 16), lambda t: (t, 0, 0)),
        ],
        out_shape=[
            jax.ShapeDtypeStruct((TBLK, BM, K), jnp.int32),
            jax.ShapeDtypeStruct((TBLK, BM, K), jnp.float32),
            jax.ShapeDtypeStruct((TBLK, 2, 16), jnp.int32),
        ],
    )(x2d, rwt)


def _sort_sc(idx16, cnt16):
    mesh = plsc.VectorSubcoreMesh(core_axis_name="c", subcore_axis_name="s")

    @functools.partial(
        pl.kernel,
        out_type=[
            jax.ShapeDtypeStruct((16, 256), jnp.int32),
            jax.ShapeDtypeStruct((PMAX,), jnp.int32),
            jax.ShapeDtypeStruct((64,), jnp.int32),
        ],
        mesh=mesh,
        scratch_types=[
            pltpu.VMEM((256,), jnp.int32),
            pltpu.VMEM((256,), jnp.int32),
            pltpu.VMEM((2, 128), jnp.int32),
            pltpu.VMEM((2, 128), jnp.int32),
            pltpu.VMEM((16, 16), jnp.int32),
            pltpu.VMEM((64,), jnp.int32),
            pltpu.SemaphoreType.DMA,
        ],
        compiler_params=_sc_compiler_params(),
    )
    def k(idx_hbm, cnt_hbm, pos_hbm, st_hbm, meta_hbm, ev, posl, posx, tvals,
          allc, metal, sem):
        cid = lax.axis_index("c")
        sid = lax.axis_index("s")
        lanes = lax.broadcasted_iota(jnp.int32, (16,), 0)

        @pl.when(cid == 0)
        def _():
            pltpu.sync_copy(idx_hbm.at[sid], ev)
            pltpu.sync_copy(cnt_hbm, allc)

            tot = jnp.zeros((16,), jnp.int32)
            bef = jnp.zeros((16,), jnp.int32)
            for w in range(16):
                row = allc[w]
                tot = tot + row
                bef = bef + row * (w < sid).astype(jnp.int32)
            padded = ((tot + (BM - 1)) >> 8) << 8
            css = plsc.cumsum(padded)
            start = css - padded
            basev = start + bef

            for v in range(16):
                evv = ev[pl.ds(v * 16, 16)]
                rank = jnp.zeros((16,), jnp.int32)
                hv = jnp.zeros((16,), jnp.int32)
                for e in range(E):
                    m = evv == e
                    cs = plsc.cumsum(m.astype(jnp.int32))
                    rank = rank + jnp.where(m, cs - 1, 0)
                    pc = plsc.all_reduce_population_count(m)
                    hv = hv + jnp.where(lanes == e, pc, 0)
                posv = basev.at[evv].get(mode="promise_in_bounds") + rank
                basev = basev + hv
                posl[pl.ds(v * 16, 16)] = posv
                posx[v // 8, pl.ds((v % 8) * 16, 16)] = posv
                jv = sid * 256 + v * 16 + lanes
                tv = ((jv >> 9) << 8) | ((jv >> 1) & 255)
                tvals[v // 8, pl.ds((v % 8) * 16, 16)] = tv

            pltpu.sync_copy(posl, pos_hbm.at[sid])
            for ch in range(2):
                pltpu.sync_copy(tvals.at[ch], st_hbm.at[posx.at[ch]])

            @pl.when(sid == 0)
            def _():
                nbt = jnp.sum(jnp.where(lanes == E - 1, css, 0)) >> 8
                for r in range(2):
                    bpos = (r * 16 + lanes) << 8
                    acc = jnp.zeros((16,), jnp.int32)
                    for e in range(E):
                        se = jnp.sum(jnp.where(lanes == e, start, 0))
                        acc = acc + (se <= bpos).astype(jnp.int32)
                    metal[pl.ds(r * 16, 16)] = jnp.clip(acc - 1, 0, E - 1)
                nbtv = jnp.zeros((16,), jnp.int32) + nbt
                metal[pl.ds(32, 16)] = nbtv
                metal[pl.ds(48, 16)] = nbtv
                pltpu.sync_copy(metal, meta_hbm)

    return k(idx16, cnt16)


def _gather_rows_sc(src, idx_flat, nrows):
    mesh = plsc.VectorSubcoreMesh(core_axis_name="c", subcore_axis_name="s")
    nsrc = src.shape[0]
    bpw = nrows // 32
    nch = 8
    ch = bpw // nch

    @functools.partial(
        pl.kernel,
        out_type=jax.ShapeDtypeStruct((nrows, H), src.dtype),
        mesh=mesh,
        scratch_types=[
            pltpu.VMEM((bpw,), jnp.int32),
            pltpu.VMEM((ch, H), jnp.float32),
            pltpu.VMEM((ch, H), jnp.float32),
            pltpu.SemaphoreType.DMA,
            pltpu.SemaphoreType.DMA,
        ],
        compiler_params=_sc_compiler_params(),
    )
    def k(src_hbm, i_hbm, o_hbm, idxv, bufa, bufb, sema, semb):
        cid = lax.axis_index("c")
        sid = lax.axis_index("s")
        wid = sid * 2 + cid
        base = wid * bpw
        pltpu.sync_copy(i_hbm.at[pl.ds(base, bpw)], idxv)
        for vv in range(bpw // 16):
            cv = idxv[pl.ds(vv * 16, 16)]
            idxv[pl.ds(vv * 16, 16)] = jnp.clip(cv, 0, nsrc - 1)
        bufs = [bufa, bufb]
        sems = [sema, semb]
        cps = [None, None]
        cps[0] = pltpu.async_copy(
            src_hbm.at[idxv.at[pl.ds(0, ch)]], bufa, sema)
        for c in range(nch):
            if c + 1 < nch:
                cps[(c + 1) % 2] = pltpu.async_copy(
                    src_hbm.at[idxv.at[pl.ds((c + 1) * ch, ch)]],
                    bufs[(c + 1) % 2], sems[(c + 1) % 2])
            cps[c % 2].wait()
            pltpu.sync_copy(bufs[c % 2], o_hbm.at[pl.ds(base + c * ch, ch)])

    return k(src, idx_flat)


def _g1a_body(meta_ref, xs_ref, w1_ref, gsx_ref, w1c, ce):
    i = pl.program_id(0)
    e = meta_ref[i]

    @pl.when(jnp.logical_or(i == 0, ce[0] != e))
    def _():
        w1c[...] = w1_ref[0].astype(jnp.bfloat16)
        ce[0] = e

    @pl.when(i < meta_ref[META_NBT])
    def _():
        xb = xs_ref[...].astype(jnp.bfloat16)
        gt = jnp.dot(xb, w1c[...], preferred_element_type=jnp.float32)
        gsx_ref[...] = (gt * lax.logistic(gt)).astype(jnp.bfloat16)


def _g1a(meta, xs, w1):
    grid_spec = pltpu.PrefetchScalarGridSpec(
        num_scalar_prefetch=1,
        grid=(NBLK,),
        in_specs=[
            pl.BlockSpec((BM, H), lambda i, m: (i, 0)),
            pl.BlockSpec((1, H, F), lambda i, m: (m[i], 0, 0)),
        ],
        out_specs=pl.BlockSpec((BM, F), lambda i, m: (i, 0)),
        scratch_shapes=[
            pltpu.VMEM((H, F), jnp.bfloat16),
            pltpu.SMEM((1,), jnp.int32),
        ],
    )
    return pl.pallas_call(
        _g1a_body,
        grid_spec=grid_spec,
        out_shape=jax.ShapeDtypeStruct((PMAX, F), jnp.bfloat16),
    )(meta, xs, w1)


def _g1b_body(meta_ref, xs_ref, w3_ref, gsx_ref, act_ref, w3c, ce):
    i = pl.program_id(0)
    e = meta_ref[i]

    @pl.when(jnp.logical_or(i == 0, ce[0] != e))
    def _():
        w3c[...] = w3_ref[0].astype(jnp.bfloat16)
        ce[0] = e

    @pl.when(i < meta_ref[META_NBT])
    def _():
        xb = xs_ref[...].astype(jnp.bfloat16)
        up = jnp.dot(xb, w3c[...], preferred_element_type=jnp.float32)
        act_ref[...] = (gsx_ref[...].astype(jnp.float32) * up).astype(
            jnp.bfloat16)


def _g1b(meta, xs, w3, gsx):
    grid_spec = pltpu.PrefetchScalarGridSpec(
        num_scalar_prefetch=1,
        grid=(NBLK,),
        in_specs=[
            pl.BlockSpec((BM, H), lambda i, m: (i, 0)),
            pl.BlockSpec((1, H, F), lambda i, m: (m[i], 0, 0)),
            pl.BlockSpec((BM, F), lambda i, m: (i, 0)),
        ],
        out_specs=pl.BlockSpec((BM, F), lambda i, m: (i, 0)),
        scratch_shapes=[
            pltpu.VMEM((H, F), jnp.bfloat16),
            pltpu.SMEM((1,), jnp.int32),
        ],
    )
    return pl.pallas_call(
        _g1b_body,
        grid_spec=grid_spec,
        out_shape=jax.ShapeDtypeStruct((PMAX, F), jnp.bfloat16),
    )(meta, xs, w3, gsx)


def _g2_body(meta_ref, act_ref, w2_ref, ys_ref, w2c, ce):
    i = pl.program_id(0)
    e = meta_ref[i]

    @pl.when(jnp.logical_or(i == 0, ce[0] != e))
    def _():
        w2c[...] = w2_ref[0].astype(jnp.bfloat16)
        ce[0] = e

    @pl.when(i < meta_ref[META_NBT])
    def _():
        ys_ref[...] = jnp.dot(act_ref[...], w2c[...],
                              preferred_element_type=jnp.float32)


def _g2(meta, act, w2):
    grid_spec = pltpu.PrefetchScalarGridSpec(
        num_scalar_prefetch=1,
        grid=(NBLK,),
        in_specs=[
            pl.BlockSpec((BM, F), lambda i, m: (i, 0)),
            pl.BlockSpec((1, F, H), lambda i, m: (m[i], 0, 0)),
        ],
        out_specs=pl.BlockSpec((BM, H), lambda i, m: (i, 0)),
        scratch_shapes=[
            pltpu.VMEM((F, H), jnp.bfloat16),
            pltpu.SMEM((1,), jnp.int32),
        ],
    )
    return pl.pallas_call(
        _g2_body,
        grid_spec=grid_spec,
        out_shape=jax.ShapeDtypeStruct((PMAX, H), jnp.float32),
    )(meta, act, w2)


def _s1a_body(x_ref, sgu_ref, gs_ref, wgc, cc):
    c = pl.program_id(0)
    t = pl.program_id(1)

    @pl.when(jnp.logical_or(jnp.logical_and(c == 0, t == 0), cc[0] != c))
    def _():
        wgc[...] = sgu_ref[...].astype(jnp.bfloat16)
        cc[0] = c

    xb = x_ref[...].astype(jnp.bfloat16)
    g = jnp.dot(xb, wgc[...], preferred_element_type=jnp.float32)
    gs_ref[...] = (g * lax.logistic(g)).astype(jnp.bfloat16)


def _s1a(x2d, sgu):
    return pl.pallas_call(
        _s1a_body,
        grid=(NIC, TBLK),
        in_specs=[
            pl.BlockSpec((BM, H), lambda c, t: (t, 0)),
            pl.BlockSpec((H, IC), lambda c, t: (0, c)),
        ],
        out_specs=pl.BlockSpec((BM, IC), lambda c, t: (t, c)),
        out_shape=jax.ShapeDtypeStruct((T, I), jnp.bfloat16),
        scratch_shapes=[
            pltpu.VMEM((H, IC), jnp.bfloat16),
            pltpu.SMEM((1,), jnp.int32),
        ],
    )(x2d, sgu)


def _s1b_body(x_ref, sgu_ref, gs_ref, act_ref, wuc, cc):
    c = pl.program_id(0)
    t = pl.program_id(1)

    @pl.when(jnp.logical_or(jnp.logical_and(c == 0, t == 0), cc[0] != c))
    def _():
        wuc[...] = sgu_ref[...].astype(jnp.bfloat16)
        cc[0] = c

    xb = x_ref[...].astype(jnp.bfloat16)
    u = jnp.dot(xb, wuc[...], preferred_element_type=jnp.float32)
    act_ref[...] = (gs_ref[...].astype(jnp.float32) * u).astype(jnp.bfloat16)


def _s1b(x2d, sgu, gs):
    return pl.pallas_call(
        _s1b_body,
        grid=(NIC, TBLK),
        in_specs=[
            pl.BlockSpec((BM, H), lambda c, t: (t, 0)),
            pl.BlockSpec((H, IC), lambda c, t: (0, NIC + c)),
            pl.BlockSpec((BM, IC), lambda c, t: (t, c)),
        ],
        out_specs=pl.BlockSpec((BM, IC), lambda c, t: (t, c)),
        out_shape=jax.ShapeDtypeStruct((T, I), jnp.bfloat16),
        scratch_shapes=[
            pltpu.VMEM((H, IC), jnp.bfloat16),
            pltpu.SMEM((1,), jnp.int32),
        ],
    )(x2d, sgu, gs)


def _s2_body(act_ref, sd_ref, g_ref, w_ref, out_ref):
    sh = jnp.dot(act_ref[...], sd_ref[...], preferred_element_type=jnp.float32)
    g0 = g_ref[0, :, 0, :]
    g1 = g_ref[0, :, 1, :]
    w0 = w_ref[0, :, 0:1]
    w1 = w_ref[0, :, 1:2]
    out_ref[...] = w0 * g0 + w1 * g1 + sh


def _s2(act, sdb, g4, wv):
    return pl.pallas_call(
        _s2_body,
        grid=(TBLK,),
        in_specs=[
            pl.BlockSpec((BM, I), lambda t: (t, 0)),
            pl.BlockSpec((I, H), lambda t: (0, 0)),
            pl.BlockSpec((1, BM, K, H), lambda t: (t, 0, 0, 0)),
            pl.BlockSpec((1, BM, K), lambda t: (t, 0, 0)),
        ],
        out_specs=pl.BlockSpec((BM, H), lambda t: (t, 0)),
        out_shape=jax.ShapeDtypeStruct((T, H), jnp.float32),
    )(act, sdb, g4, wv)


def kernel(hidden_states, router_weight, w1, w3, w2, shared_gate_up,
           shared_down):
    b, s, h = hidden_states.shape
    x2d = hidden_states.reshape(T, H)
    rwt = router_weight.T

    idx, wv, cnt = _router(x2d, rwt)
    idx16 = idx.reshape(16, 256)

    pos, sort_tok, meta = _sort_sc(idx16, cnt.reshape(16, 16))

    gs = _s1a(x2d, shared_gate_up)
    act_sh = _s1b(x2d, shared_gate_up, gs)

    xs = _gather_rows_sc(x2d, sort_tok, PMAX)
    gsx = _g1a(meta, xs, w1)
    act = _g1b(meta, xs, w3, gsx)
    ys = _g2(meta, act, w2)
    g = _gather_rows_sc(ys, pos.reshape(-1), T * K)
    g4 = g.reshape(TBLK, BM, K, H)

    out = _s2(act_sh, shared_down.astype(jnp.bfloat16), g4, wv)
    return out.reshape(b, s, h)


# speedup vs baseline: 5.8037x; 1.2359x over previous
import dataclasses
import functools

import jax
import jax.numpy as jnp
from jax import lax
from jax.experimental import pallas as pl
from jax.experimental.pallas import tpu as pltpu
from jax.experimental.pallas import tpu_sc as plsc

T, H, E, K, F, I = 2048, 2048, 8, 2, 1664, 3328
BM = 256
PMAX = T * K + E * BM
NBLK = PMAX // BM
TBLK = T // BM
H2 = H // 2
IC = I // 2
NIC = 2
GW = 16
META_NBT = 32
NEG = -1e30


def _sc_compiler_params():
    cp = pltpu.CompilerParams()
    if "needs_layout_passes" in pltpu.CompilerParams.__dataclass_fields__:
        cp = dataclasses.replace(cp, needs_layout_passes=False)
    return cp


def _router_body(x_ref, rwt_ref, idx_ref, w_ref, cnt_ref):
    xb = x_ref[...]
    logits = jnp.dot(xb, rwt_ref[...], preferred_element_type=jnp.float32)
    eio = lax.broadcasted_iota(jnp.int32, (BM, E), 1)
    m1 = jnp.max(logits, axis=1, keepdims=True)
    am1 = jnp.min(jnp.where(logits == m1, eio, E), axis=1, keepdims=True)
    l2 = jnp.where(eio == am1, NEG, logits)
    m2 = jnp.max(l2, axis=1, keepdims=True)
    am2 = jnp.min(jnp.where(l2 == m2, eio, E), axis=1, keepdims=True)
    ew = jnp.exp(m2 - m1)
    w2v = ew / (1.0 + ew)
    w1v = 1.0 - w2v
    idx_ref[0] = jnp.concatenate([am1, am2], axis=1)
    w_ref[0] = jnp.concatenate([w1v, w2v], axis=1)
    oh = (eio == am1).astype(jnp.int32) + (eio == am2).astype(jnp.int32)
    rio = lax.broadcasted_iota(jnp.int32, (BM, E), 0)
    h0 = jnp.sum(jnp.where(rio < BM // 2, oh, 0), axis=0, keepdims=True)
    h1 = jnp.sum(jnp.where(rio >= BM // 2, oh, 0), axis=0, keepdims=True)
    z = jnp.zeros((1, E), jnp.int32)
    cnt_ref[0] = jnp.concatenate(
        [jnp.concatenate([h0, z], axis=1),
         jnp.concatenate([h1, z], axis=1)], axis=0)


def _router(x2d, rwt):
    return pl.pallas_call(
        _router_body,
        grid=(TBLK,),
        in_specs=[
            pl.BlockSpec((BM, H), lambda t: (t, 0)),
            pl.BlockSpec((H, E), lambda t: (0, 0)),
        ],
        out_specs=[
            pl.BlockSpec((1, BM, K), lambda t: (t, 0, 0)),
            pl.BlockSpec((1, BM, K), lambda t: (t, 0, 0)),
            pl.BlockSpec((1, 2, 16), lambda t: (t, 0, 0)),
        ],
        out_shape=[
            jax.ShapeDtypeStruct((TBLK, BM, K), jnp.int32),
            jax.ShapeDtypeStruct((TBLK, BM, K), jnp.float32),
            jax.ShapeDtypeStruct((TBLK, 2, 16), jnp.int32),
        ],
    )(x2d, rwt)


def _sort_sc(idx16, cnt16):
    mesh = plsc.VectorSubcoreMesh(core_axis_name="c", subcore_axis_name="s")

    @functools.partial(
        pl.kernel,
        out_type=[
            jax.ShapeDtypeStruct((16, 256), jnp.int32),
            jax.ShapeDtypeStruct((PMAX,), jnp.int32),
            jax.ShapeDtypeStruct((64,), jnp.int32),
            jax.ShapeDtypeStruct((2, 16, 128), jnp.int32),
        ],
        mesh=mesh,
        scratch_types=[
            pltpu.VMEM((256,), jnp.int32),
            pltpu.VMEM((256,), jnp.int32),
            pltpu.VMEM((2, 128), jnp.int32),
            pltpu.VMEM((2, 128), jnp.int32),
            pltpu.VMEM((16, 16), jnp.int32),
            pltpu.VMEM((64,), jnp.int32),
            pltpu.VMEM((256,), jnp.int32),
            pltpu.SemaphoreType.DMA,
        ],
        compiler_params=_sc_compiler_params(),
    )
    def k(idx_hbm, cnt_hbm, pos_hbm, st_hbm, meta_hbm, posk_hbm, ev, posl,
          posx, tvals, allc, metal, pk, sem):
        cid = lax.axis_index("c")
        sid = lax.axis_index("s")
        lanes = lax.broadcasted_iota(jnp.int32, (16,), 0)

        @pl.when(cid == 0)
        def _():
            pltpu.sync_copy(idx_hbm.at[sid], ev)
            pltpu.sync_copy(cnt_hbm, allc)

            tot = jnp.zeros((16,), jnp.int32)
            bef = jnp.zeros((16,), jnp.int32)
            for w in range(16):
                row = allc[w]
                tot = tot + row
                bef = bef + row * (w < sid).astype(jnp.int32)
            padded = ((tot + (BM - 1)) >> 8) << 8
            css = plsc.cumsum(padded)
            start = css - padded
            basev = start + bef

            for v in range(16):
                evv = ev[pl.ds(v * 16, 16)]
                rank = jnp.zeros((16,), jnp.int32)
                hv = jnp.zeros((16,), jnp.int32)
                for e in range(E):
                    m = evv == e
                    cs = plsc.cumsum(m.astype(jnp.int32))
                    rank = rank + jnp.where(m, cs - 1, 0)
                    pc = plsc.all_reduce_population_count(m)
                    hv = hv + jnp.where(lanes == e, pc, 0)
                posv = basev.at[evv].get(mode="promise_in_bounds") + rank
                basev = basev + hv
                posl[pl.ds(v * 16, 16)] = posv
                posx[v // 8, pl.ds((v % 8) * 16, 16)] = posv
                tgtv = (lanes & 1) * 128 + v * 8 + (lanes >> 1)
                plsc.store_scatter(pk, [tgtv], posv)
                jv = sid * 256 + v * 16 + lanes
                tv = ((jv >> 9) << 8) | ((jv >> 1) & 255)
                tvals[v // 8, pl.ds((v % 8) * 16, 16)] = tv

            pltpu.sync_copy(posl, pos_hbm.at[sid])
            pltpu.sync_copy(pk.at[pl.ds(0, 128)], posk_hbm.at[0, sid])
            pltpu.sync_copy(pk.at[pl.ds(128, 128)], posk_hbm.at[1, sid])
            for ch in range(2):
                pltpu.sync_copy(tvals.at[ch], st_hbm.at[posx.at[ch]])

            @pl.when(sid == 0)
            def _():
                nbt = jnp.sum(jnp.where(lanes == E - 1, css, 0)) >> 8
                for r in range(2):
                    bpos = (r * 16 + lanes) << 8
                    acc = jnp.zeros((16,), jnp.int32)
                    for e in range(E):
                        se = jnp.sum(jnp.where(lanes == e, start, 0))
                        acc = acc + (se <= bpos).astype(jnp.int32)
                    metal[pl.ds(r * 16, 16)] = jnp.clip(acc - 1, 0, E - 1)
                nbtv = jnp.zeros((16,), jnp.int32) + nbt
                metal[pl.ds(32, 16)] = nbtv
                metal[pl.ds(48, 16)] = nbtv
                pltpu.sync_copy(metal, meta_hbm)

    return k(idx16, cnt16)


def _gather_rows_sc(src, idx_flat, nrows):
    mesh = plsc.VectorSubcoreMesh(core_axis_name="c", subcore_axis_name="s")
    nsrc, ncol = src.shape
    bpw = nrows // 32
    nch = 8
    ch = bpw // nch

    @functools.partial(
        pl.kernel,
        out_type=jax.ShapeDtypeStruct((nrows, ncol), src.dtype),
        mesh=mesh,
        scratch_types=[
            pltpu.VMEM((bpw,), jnp.int32),
            pltpu.VMEM((ch, ncol), src.dtype),
            pltpu.VMEM((ch, ncol), src.dtype),
            pltpu.SemaphoreType.DMA,
            pltpu.SemaphoreType.DMA,
        ],
        compiler_params=_sc_compiler_params(),
    )
    def k(src_hbm, i_hbm, o_hbm, idxv, bufa, bufb, sema, semb):
        cid = lax.axis_index("c")
        sid = lax.axis_index("s")
        wid = sid * 2 + cid
        base = wid * bpw
        pltpu.sync_copy(i_hbm.at[pl.ds(base, bpw)], idxv)
        for vv in range(bpw // 16):
            cv = idxv[pl.ds(vv * 16, 16)]
            idxv[pl.ds(vv * 16, 16)] = jnp.clip(cv, 0, nsrc - 1)
        bufs = [bufa, bufb]
        sems = [sema, semb]
        cps = [None, None]
        cps[0] = pltpu.async_copy(
            src_hbm.at[idxv.at[pl.ds(0, ch)]], bufa, sema)
        for c in range(nch):
            if c + 1 < nch:
                cps[(c + 1) % 2] = pltpu.async_copy(
                    src_hbm.at[idxv.at[pl.ds((c + 1) * ch, ch)]],
                    bufs[(c + 1) % 2], sems[(c + 1) % 2])
            cps[c % 2].wait()
            pltpu.sync_copy(bufs[c % 2], o_hbm.at[pl.ds(base + c * ch, ch)])

    return k(src, idx_flat)


def _scatter_rows_sc(src, posk):
    mesh = plsc.VectorSubcoreMesh(core_axis_name="c", subcore_axis_name="s")

    @functools.partial(
        pl.kernel,
        out_type=jax.ShapeDtypeStruct((PMAX, H), jnp.float32),
        mesh=mesh,
        scratch_types=[
            pltpu.VMEM((32, H), jnp.float32),
            pltpu.VMEM((4, 32), jnp.int32),
            pltpu.SemaphoreType.DMA,
        ],
        compiler_params=_sc_compiler_params(),
    )
    def k(src_hbm, pk_hbm, xs_hbm, xbuf, pk, sem):
        cid = lax.axis_index("c")
        sid = lax.axis_index("s")
        wid = sid * 2 + cid
        srow = wid // 2
        soff = (wid % 2) * 64
        for kk in range(2):
            for c in range(2):
                pltpu.sync_copy(
                    pk_hbm.at[kk, srow, pl.ds(soff + c * 32, 32)],
                    pk.at[kk * 2 + c])
        for c in range(2):
            pltpu.sync_copy(src_hbm.at[pl.ds(wid * 64 + c * 32, 32)], xbuf)
            pltpu.sync_copy(xbuf, xs_hbm.at[pk.at[c]])
            pltpu.sync_copy(xbuf, xs_hbm.at[pk.at[2 + c]])

    return k(src, posk)


def _g1a_body(meta_ref, xs_ref, w1_ref, gsx_ref, w1c, ce):
    i = pl.program_id(0)
    e = meta_ref[i]

    @pl.when(jnp.logical_or(i == 0, ce[0] != e))
    def _():
        w1c[...] = w1_ref[0].astype(jnp.bfloat16)
        ce[0] = e

    @pl.when(i < meta_ref[META_NBT])
    def _():
        xb = xs_ref[...].astype(jnp.bfloat16)
        gt = jnp.dot(xb, w1c[...], preferred_element_type=jnp.float32)
        gsx_ref[...] = (gt * lax.logistic(gt)).astype(jnp.bfloat16)


def _g1a(meta, xs, w1):
    grid_spec = pltpu.PrefetchScalarGridSpec(
        num_scalar_prefetch=1,
        grid=(NBLK,),
        in_specs=[
            pl.BlockSpec((BM, H), lambda i, m: (i, 0)),
            pl.BlockSpec((1, H, F), lambda i, m: (m[i], 0, 0)),
        ],
        out_specs=pl.BlockSpec((BM, F), lambda i, m: (i, 0)),
        scratch_shapes=[
            pltpu.VMEM((H, F), jnp.bfloat16),
            pltpu.SMEM((1,), jnp.int32),
        ],
    )
    return pl.pallas_call(
        _g1a_body,
        grid_spec=grid_spec,
        out_shape=jax.ShapeDtypeStruct((PMAX, F), jnp.bfloat16),
    )(meta, xs, w1)


def _g1b_body(meta_ref, xs_ref, w3_ref, gsx_ref, act_ref, w3c, ce):
    i = pl.program_id(0)
    e = meta_ref[i]

    @pl.when(jnp.logical_or(i == 0, ce[0] != e))
    def _():
        w3c[...] = w3_ref[0].astype(jnp.bfloat16)
        ce[0] = e

    @pl.when(i < meta_ref[META_NBT])
    def _():
        xb = xs_ref[...].astype(jnp.bfloat16)
        up = jnp.dot(xb, w3c[...], preferred_element_type=jnp.float32)
        act_ref[...] = (gsx_ref[...].astype(jnp.float32) * up).astype(
            jnp.bfloat16)


def _g1b(meta, xs, w3, gsx):
    grid_spec = pltpu.PrefetchScalarGridSpec(
        num_scalar_prefetch=1,
        grid=(NBLK,),
        in_specs=[
            pl.BlockSpec((BM, H), lambda i, m: (i, 0)),
            pl.BlockSpec((1, H, F), lambda i, m: (m[i], 0, 0)),
            pl.BlockSpec((BM, F), lambda i, m: (i, 0)),
        ],
        out_specs=pl.BlockSpec((BM, F), lambda i, m: (i, 0)),
        scratch_shapes=[
            pltpu.VMEM((H, F), jnp.bfloat16),
            pltpu.SMEM((1,), jnp.int32),
        ],
    )
    return pl.pallas_call(
        _g1b_body,
        grid_spec=grid_spec,
        out_shape=jax.ShapeDtypeStruct((PMAX, F), jnp.bfloat16),
    )(meta, xs, w3, gsx)


def _g2_body(meta_ref, act_ref, w2_ref, ys_ref, w2c, ce):
    i = pl.program_id(0)
    e = meta_ref[i]

    @pl.when(jnp.logical_or(i == 0, ce[0] != e))
    def _():
        w2c[...] = w2_ref[0].astype(jnp.bfloat16)
        ce[0] = e

    @pl.when(i < meta_ref[META_NBT])
    def _():
        ys_ref[...] = jnp.dot(act_ref[...], w2c[...],
                              preferred_element_type=jnp.float32)


def _g2(meta, act, w2):
    grid_spec = pltpu.PrefetchScalarGridSpec(
        num_scalar_prefetch=1,
        grid=(NBLK,),
        in_specs=[
            pl.BlockSpec((BM, F), lambda i, m: (i, 0)),
            pl.BlockSpec((1, F, H), lambda i, m: (m[i], 0, 0)),
        ],
        out_specs=pl.BlockSpec((BM, H), lambda i, m: (i, 0)),
        scratch_shapes=[
            pltpu.VMEM((F, H), jnp.bfloat16),
            pltpu.SMEM((1,), jnp.int32),
        ],
    )
    return pl.pallas_call(
        _g2_body,
        grid_spec=grid_spec,
        out_shape=jax.ShapeDtypeStruct((PMAX, H), jnp.float32),
    )(meta, act, w2)


def _s1a_body(x_ref, sgu_ref, gs_ref, wgc, cc):
    c = pl.program_id(0)
    t = pl.program_id(1)

    @pl.when(jnp.logical_or(jnp.logical_and(c == 0, t == 0), cc[0] != c))
    def _():
        wgc[...] = sgu_ref[...].astype(jnp.bfloat16)
        cc[0] = c

    xb = x_ref[...].astype(jnp.bfloat16)
    g = jnp.dot(xb, wgc[...], preferred_element_type=jnp.float32)
    gs_ref[...] = (g * lax.logistic(g)).astype(jnp.bfloat16)


def _s1a(x2d, sgu):
    return pl.pallas_call(
        _s1a_body,
        grid=(NIC, TBLK),
        in_specs=[
            pl.BlockSpec((BM, H), lambda c, t: (t, 0)),
            pl.BlockSpec((H, IC), lambda c, t: (0, c)),
        ],
        out_specs=pl.BlockSpec((BM, IC), lambda c, t: (t, c)),
        out_shape=jax.ShapeDtypeStruct((T, I), jnp.bfloat16),
        scratch_shapes=[
            pltpu.VMEM((H, IC), jnp.bfloat16),
            pltpu.SMEM((1,), jnp.int32),
        ],
    )(x2d, sgu)


def _s1b_body(x_ref, sgu_ref, gs_ref, act_ref, wuc, cc):
    c = pl.program_id(0)
    t = pl.program_id(1)

    @pl.when(jnp.logical_or(jnp.logical_and(c == 0, t == 0), cc[0] != c))
    def _():
        wuc[...] = sgu_ref[...].astype(jnp.bfloat16)
        cc[0] = c

    xb = x_ref[...].astype(jnp.bfloat16)
    u = jnp.dot(xb, wuc[...], preferred_element_type=jnp.float32)
    act_ref[...] = (gs_ref[...].astype(jnp.float32) * u).astype(jnp.bfloat16)


def _s1b(x2d, sgu, gs):
    return pl.pallas_call(
        _s1b_body,
        grid=(NIC, TBLK),
        in_specs=[
            pl.BlockSpec((BM, H), lambda c, t: (t, 0)),
            pl.BlockSpec((H, IC), lambda c, t: (0, NIC + c)),
            pl.BlockSpec((BM, IC), lambda c, t: (t, c)),
        ],
        out_specs=pl.BlockSpec((BM, IC), lambda c, t: (t, c)),
        out_shape=jax.ShapeDtypeStruct((T, I), jnp.bfloat16),
        scratch_shapes=[
            pltpu.VMEM((H, IC), jnp.bfloat16),
            pltpu.SMEM((1,), jnp.int32),
        ],
    )(x2d, sgu, gs)


def _s2_body(act_ref, sd_ref, g_ref, w_ref, out_ref):
    sh = jnp.dot(act_ref[...], sd_ref[...], preferred_element_type=jnp.float32)
    g0 = g_ref[0, :, 0, :]
    g1 = g_ref[0, :, 1, :]
    w0 = w_ref[0, :, 0:1]
    w1 = w_ref[0, :, 1:2]
    out_ref[...] = w0 * g0 + w1 * g1 + sh


def _s2(act, sdb, g4, wv):
    return pl.pallas_call(
        _s2_body,
        grid=(TBLK,),
        in_specs=[
            pl.BlockSpec((BM, I), lambda t: (t, 0)),
            pl.BlockSpec((I, H), lambda t: (0, 0)),
            pl.BlockSpec((1, BM, K, H), lambda t: (t, 0, 0, 0)),
            pl.BlockSpec((1, BM, K), lambda t: (t, 0, 0)),
        ],
        out_specs=pl.BlockSpec((BM, H), lambda t: (t, 0)),
        out_shape=jax.ShapeDtypeStruct((T, H), jnp.float32),
    )(act, sdb, g4, wv)


def kernel(hidden_states, router_weight, w1, w3, w2, shared_gate_up,
           shared_down):
    b, s, h = hidden_states.shape
    x2d = hidden_states.reshape(T, H)
    rwt = router_weight.T

    idx, wv, cnt = _router(x2d, rwt)
    idx16 = idx.reshape(16, 256)

    pos, sort_tok, meta, posk = _sort_sc(idx16, cnt.reshape(16, 16))

    gs = _s1a(x2d, shared_gate_up)
    act_sh = _s1b(x2d, shared_gate_up, gs)

    xs = _scatter_rows_sc(x2d, posk)
    gsx = _g1a(meta, xs, w1)
    act = _g1b(meta, xs, w3, gsx)
    ys = _g2(meta, act, w2)
    g = _gather_rows_sc(ys, pos.reshape(-1), T * K)
    g4 = g.reshape(TBLK, BM, K, H)

    out = _s2(act_sh, shared_down.astype(jnp.bfloat16), g4, wv)
    return out.reshape(b, s, h)


# speedup vs baseline: 6.0802x; 1.0476x over previous
import dataclasses
import functools

import jax
import jax.numpy as jnp
from jax import lax
from jax.experimental import pallas as pl
from jax.experimental.pallas import tpu as pltpu
from jax.experimental.pallas import tpu_sc as plsc

T, H, E, K, F, I = 2048, 2048, 8, 2, 1664, 3328
BM = 256
PMAX = T * K + E * BM
NBLK = PMAX // BM
TBLK = T // BM
H2 = H // 2
IC = I // 2
NIC = 2
GW = 16
META_NBT = 32
NEG = -1e30


def _sc_compiler_params():
    cp = pltpu.CompilerParams()
    if "needs_layout_passes" in pltpu.CompilerParams.__dataclass_fields__:
        cp = dataclasses.replace(cp, needs_layout_passes=False)
    return cp


def _router_body(x_ref, rwt_ref, idx_ref, w_ref, cnt_ref):
    xb = x_ref[...]
    logits = jnp.dot(xb, rwt_ref[...], preferred_element_type=jnp.float32)
    eio = lax.broadcasted_iota(jnp.int32, (BM, E), 1)
    m1 = jnp.max(logits, axis=1, keepdims=True)
    am1 = jnp.min(jnp.where(logits == m1, eio, E), axis=1, keepdims=True)
    l2 = jnp.where(eio == am1, NEG, logits)
    m2 = jnp.max(l2, axis=1, keepdims=True)
    am2 = jnp.min(jnp.where(l2 == m2, eio, E), axis=1, keepdims=True)
    ew = jnp.exp(m2 - m1)
    w2v = ew / (1.0 + ew)
    w1v = 1.0 - w2v
    idx_ref[0] = jnp.concatenate([am1, am2], axis=1)
    w_ref[0] = jnp.concatenate([w1v, w2v], axis=1)
    oh = (eio == am1).astype(jnp.int32) + (eio == am2).astype(jnp.int32)
    rio = lax.broadcasted_iota(jnp.int32, (BM, E), 0)
    h0 = jnp.sum(jnp.where(rio < BM // 2, oh, 0), axis=0, keepdims=True)
    h1 = jnp.sum(jnp.where(rio >= BM // 2, oh, 0), axis=0, keepdims=True)
    z = jnp.zeros((1, E), jnp.int32)
    cnt_ref[0] = jnp.concatenate(
        [jnp.concatenate([h0, z], axis=1),
         jnp.concatenate([h1, z], axis=1)], axis=0)


def _router(x2d, rwt):
    return pl.pallas_call(
        _router_body,
        grid=(TBLK,),
        in_specs=[
            pl.BlockSpec((BM, H), lambda t: (t, 0)),
            pl.BlockSpec((H, E), lambda t: (0, 0)),
        ],
        out_specs=[
            pl.BlockSpec((1, BM, K), lambda t: (t, 0, 0)),
            pl.BlockSpec((1, BM, K), lambda t: (t, 0, 0)),
            pl.BlockSpec((1, 2, 16), lambda t: (t, 0, 0)),
        ],
        out_shape=[
            jax.ShapeDtypeStruct((TBLK, BM, K), jnp.int32),
            jax.ShapeDtypeStruct((TBLK, BM, K), jnp.float32),
            jax.ShapeDtypeStruct((TBLK, 2, 16), jnp.int32),
        ],
    )(x2d, rwt)


def _sort_sc(idx16, cnt16):
    mesh = plsc.VectorSubcoreMesh(core_axis_name="c", subcore_axis_name="s")

    @functools.partial(
        pl.kernel,
        out_type=[
            jax.ShapeDtypeStruct((16, 256), jnp.int32),
            jax.ShapeDtypeStruct((64,), jnp.int32),
            jax.ShapeDtypeStruct((2, 16, 128), jnp.int32),
        ],
        mesh=mesh,
        scratch_types=[
            pltpu.VMEM((256,), jnp.int32),
            pltpu.VMEM((256,), jnp.int32),
            pltpu.VMEM((16, 16), jnp.int32),
            pltpu.VMEM((64,), jnp.int32),
            pltpu.VMEM((256,), jnp.int32),
            pltpu.SemaphoreType.DMA,
        ],
        compiler_params=_sc_compiler_params(),
    )
    def k(idx_hbm, cnt_hbm, pos_hbm, meta_hbm, posk_hbm, ev, posl,
          allc, metal, pk, sem):
        cid = lax.axis_index("c")
        sid = lax.axis_index("s")
        lanes = lax.broadcasted_iota(jnp.int32, (16,), 0)

        @pl.when(cid == 0)
        def _():
            pltpu.sync_copy(idx_hbm.at[sid], ev)
            pltpu.sync_copy(cnt_hbm, allc)

            tot = jnp.zeros((16,), jnp.int32)
            bef = jnp.zeros((16,), jnp.int32)
            for w in range(16):
                row = allc[w]
                tot = tot + row
                bef = bef + row * (w < sid).astype(jnp.int32)
            padded = ((tot + (BM - 1)) >> 8) << 8
            css = plsc.cumsum(padded)
            start = css - padded
            basev = start + bef

            for v in range(16):
                evv = ev[pl.ds(v * 16, 16)]
                rank = jnp.zeros((16,), jnp.int32)
                hv = jnp.zeros((16,), jnp.int32)
                for e in range(E):
                    m = evv == e
                    cs = plsc.cumsum(m.astype(jnp.int32))
                    rank = rank + jnp.where(m, cs - 1, 0)
                    pc = plsc.all_reduce_population_count(m)
                    hv = hv + jnp.where(lanes == e, pc, 0)
                posv = basev.at[evv].get(mode="promise_in_bounds") + rank
                basev = basev + hv
                posl[pl.ds(v * 16, 16)] = posv
                tgtv = (lanes & 1) * 128 + v * 8 + (lanes >> 1)
                plsc.store_scatter(pk, [tgtv], posv)

            pltpu.sync_copy(posl, pos_hbm.at[sid])
            pltpu.sync_copy(pk.at[pl.ds(0, 128)], posk_hbm.at[0, sid])
            pltpu.sync_copy(pk.at[pl.ds(128, 128)], posk_hbm.at[1, sid])

            @pl.when(sid == 0)
            def _():
                nbt = jnp.sum(jnp.where(lanes == E - 1, css, 0)) >> 8
                for r in range(2):
                    bpos = (r * 16 + lanes) << 8
                    acc = jnp.zeros((16,), jnp.int32)
                    for e in range(E):
                        se = jnp.sum(jnp.where(lanes == e, start, 0))
                        acc = acc + (se <= bpos).astype(jnp.int32)
                    metal[pl.ds(r * 16, 16)] = jnp.clip(acc - 1, 0, E - 1)
                nbtv = jnp.zeros((16,), jnp.int32) + nbt
                metal[pl.ds(32, 16)] = nbtv
                metal[pl.ds(48, 16)] = nbtv
                pltpu.sync_copy(metal, meta_hbm)

    return k(idx16, cnt16)


def _gather_rows_sc(src, idx_flat, nrows):
    mesh = plsc.VectorSubcoreMesh(core_axis_name="c", subcore_axis_name="s")
    nsrc, ncol = src.shape
    bpw = nrows // 32
    nch = 8
    ch = bpw // nch

    @functools.partial(
        pl.kernel,
        out_type=jax.ShapeDtypeStruct((nrows, ncol), src.dtype),
        mesh=mesh,
        scratch_types=[
            pltpu.VMEM((bpw,), jnp.int32),
            pltpu.VMEM((ch, ncol), src.dtype),
            pltpu.VMEM((ch, ncol), src.dtype),
            pltpu.SemaphoreType.DMA,
            pltpu.SemaphoreType.DMA,
        ],
        compiler_params=_sc_compiler_params(),
    )
    def k(src_hbm, i_hbm, o_hbm, idxv, bufa, bufb, sema, semb):
        cid = lax.axis_index("c")
        sid = lax.axis_index("s")
        wid = sid * 2 + cid
        base = wid * bpw
        pltpu.sync_copy(i_hbm.at[pl.ds(base, bpw)], idxv)
        for vv in range(bpw // 16):
            cv = idxv[pl.ds(vv * 16, 16)]
            idxv[pl.ds(vv * 16, 16)] = jnp.clip(cv, 0, nsrc - 1)
        bufs = [bufa, bufb]
        sems = [sema, semb]
        cps = [None, None]
        cps[0] = pltpu.async_copy(
            src_hbm.at[idxv.at[pl.ds(0, ch)]], bufa, sema)
        for c in range(nch):
            if c + 1 < nch:
                cps[(c + 1) % 2] = pltpu.async_copy(
                    src_hbm.at[idxv.at[pl.ds((c + 1) * ch, ch)]],
                    bufs[(c + 1) % 2], sems[(c + 1) % 2])
            cps[c % 2].wait()
            pltpu.sync_copy(bufs[c % 2], o_hbm.at[pl.ds(base + c * ch, ch)])

    return k(src, idx_flat)


def _scatter_rows_sc(src, posk):
    mesh = plsc.VectorSubcoreMesh(core_axis_name="c", subcore_axis_name="s")

    @functools.partial(
        pl.kernel,
        out_type=jax.ShapeDtypeStruct((PMAX, H), jnp.float32),
        mesh=mesh,
        scratch_types=[
            pltpu.VMEM((32, H), jnp.float32),
            pltpu.VMEM((4, 32), jnp.int32),
            pltpu.SemaphoreType.DMA,
        ],
        compiler_params=_sc_compiler_params(),
    )
    def k(src_hbm, pk_hbm, xs_hbm, xbuf, pk, sem):
        cid = lax.axis_index("c")
        sid = lax.axis_index("s")
        wid = sid * 2 + cid
        srow = wid // 2
        soff = (wid % 2) * 64
        for kk in range(2):
            for c in range(2):
                pltpu.sync_copy(
                    pk_hbm.at[kk, srow, pl.ds(soff + c * 32, 32)],
                    pk.at[kk * 2 + c])
        for c in range(2):
            pltpu.sync_copy(src_hbm.at[pl.ds(wid * 64 + c * 32, 32)], xbuf)
            pltpu.sync_copy(xbuf, xs_hbm.at[pk.at[c]])
            pltpu.sync_copy(xbuf, xs_hbm.at[pk.at[2 + c]])

    return k(src, posk)


def _g1a_body(meta_ref, xs_ref, w1_ref, gsx_ref, w1c, ce):
    i = pl.program_id(0)
    e = meta_ref[i]

    @pl.when(jnp.logical_or(i == 0, ce[0] != e))
    def _():
        w1c[...] = w1_ref[0].astype(jnp.bfloat16)
        ce[0] = e

    @pl.when(i < meta_ref[META_NBT])
    def _():
        xb = xs_ref[...].astype(jnp.bfloat16)
        gt = jnp.dot(xb, w1c[...], preferred_element_type=jnp.float32)
        gsx_ref[...] = (gt * lax.logistic(gt)).astype(jnp.bfloat16)


def _g1a(meta, xs, w1):
    grid_spec = pltpu.PrefetchScalarGridSpec(
        num_scalar_prefetch=1,
        grid=(NBLK,),
        in_specs=[
            pl.BlockSpec((BM, H), lambda i, m: (i, 0)),
            pl.BlockSpec((1, H, F), lambda i, m: (m[i], 0, 0)),
        ],
        out_specs=pl.BlockSpec((BM, F), lambda i, m: (i, 0)),
        scratch_shapes=[
            pltpu.VMEM((H, F), jnp.bfloat16),
            pltpu.SMEM((1,), jnp.int32),
        ],
    )
    return pl.pallas_call(
        _g1a_body,
        grid_spec=grid_spec,
        out_shape=jax.ShapeDtypeStruct((PMAX, F), jnp.bfloat16),
    )(meta, xs, w1)


def _g1b_body(meta_ref, xs_ref, w3_ref, gsx_ref, act_ref, w3c, ce):
    i = pl.program_id(0)
    e = meta_ref[i]

    @pl.when(jnp.logical_or(i == 0, ce[0] != e))
    def _():
        w3c[...] = w3_ref[0].astype(jnp.bfloat16)
        ce[0] = e

    @pl.when(i < meta_ref[META_NBT])
    def _():
        xb = xs_ref[...].astype(jnp.bfloat16)
        up = jnp.dot(xb, w3c[...], preferred_element_type=jnp.float32)
        act_ref[...] = (gsx_ref[...].astype(jnp.float32) * up).astype(
            jnp.bfloat16)


def _g1b(meta, xs, w3, gsx):
    grid_spec = pltpu.PrefetchScalarGridSpec(
        num_scalar_prefetch=1,
        grid=(NBLK,),
        in_specs=[
            pl.BlockSpec((BM, H), lambda i, m: (i, 0)),
            pl.BlockSpec((1, H, F), lambda i, m: (m[i], 0, 0)),
            pl.BlockSpec((BM, F), lambda i, m: (i, 0)),
        ],
        out_specs=pl.BlockSpec((BM, F), lambda i, m: (i, 0)),
        scratch_shapes=[
            pltpu.VMEM((H, F), jnp.bfloat16),
            pltpu.SMEM((1,), jnp.int32),
        ],
    )
    return pl.pallas_call(
        _g1b_body,
        grid_spec=grid_spec,
        out_shape=jax.ShapeDtypeStruct((PMAX, F), jnp.bfloat16),
    )(meta, xs, w3, gsx)


def _g2_body(meta_ref, act_ref, w2_ref, ys_ref, w2c, ce):
    i = pl.program_id(0)
    e = meta_ref[i]

    @pl.when(jnp.logical_or(i == 0, ce[0] != e))
    def _():
        w2c[...] = w2_ref[0].astype(jnp.bfloat16)
        ce[0] = e

    @pl.when(i < meta_ref[META_NBT])
    def _():
        ys_ref[...] = jnp.dot(act_ref[...], w2c[...],
                              preferred_element_type=jnp.float32)


def _g2(meta, act, w2):
    grid_spec = pltpu.PrefetchScalarGridSpec(
        num_scalar_prefetch=1,
        grid=(NBLK,),
        in_specs=[
            pl.BlockSpec((BM, F), lambda i, m: (i, 0)),
            pl.BlockSpec((1, F, H), lambda i, m: (m[i], 0, 0)),
        ],
        out_specs=pl.BlockSpec((BM, H), lambda i, m: (i, 0)),
        scratch_shapes=[
            pltpu.VMEM((F, H), jnp.bfloat16),
            pltpu.SMEM((1,), jnp.int32),
        ],
    )
    return pl.pallas_call(
        _g2_body,
        grid_spec=grid_spec,
        out_shape=jax.ShapeDtypeStruct((PMAX, H), jnp.float32),
    )(meta, act, w2)


def _s1a_body(x_ref, sgu_ref, gs_ref, wgc, cc):
    c = pl.program_id(0)
    t = pl.program_id(1)

    @pl.when(jnp.logical_or(jnp.logical_and(c == 0, t == 0), cc[0] != c))
    def _():
        wgc[...] = sgu_ref[...].astype(jnp.bfloat16)
        cc[0] = c

    xb = x_ref[...].astype(jnp.bfloat16)
    g = jnp.dot(xb, wgc[...], preferred_element_type=jnp.float32)
    gs_ref[...] = (g * lax.logistic(g)).astype(jnp.bfloat16)


def _s1a(x2d, sgu):
    return pl.pallas_call(
        _s1a_body,
        grid=(NIC, TBLK),
        in_specs=[
            pl.BlockSpec((BM, H), lambda c, t: (t, 0)),
            pl.BlockSpec((H, IC), lambda c, t: (0, c)),
        ],
        out_specs=pl.BlockSpec((BM, IC), lambda c, t: (t, c)),
        out_shape=jax.ShapeDtypeStruct((T, I), jnp.bfloat16),
        scratch_shapes=[
            pltpu.VMEM((H, IC), jnp.bfloat16),
            pltpu.SMEM((1,), jnp.int32),
        ],
    )(x2d, sgu)


def _s1b_body(x_ref, sgu_ref, gs_ref, act_ref, wuc, cc):
    c = pl.program_id(0)
    t = pl.program_id(1)

    @pl.when(jnp.logical_or(jnp.logical_and(c == 0, t == 0), cc[0] != c))
    def _():
        wuc[...] = sgu_ref[...].astype(jnp.bfloat16)
        cc[0] = c

    xb = x_ref[...].astype(jnp.bfloat16)
    u = jnp.dot(xb, wuc[...], preferred_element_type=jnp.float32)
    act_ref[...] = (gs_ref[...].astype(jnp.float32) * u).astype(jnp.bfloat16)


def _s1b(x2d, sgu, gs):
    return pl.pallas_call(
        _s1b_body,
        grid=(NIC, TBLK),
        in_specs=[
            pl.BlockSpec((BM, H), lambda c, t: (t, 0)),
            pl.BlockSpec((H, IC), lambda c, t: (0, NIC + c)),
            pl.BlockSpec((BM, IC), lambda c, t: (t, c)),
        ],
        out_specs=pl.BlockSpec((BM, IC), lambda c, t: (t, c)),
        out_shape=jax.ShapeDtypeStruct((T, I), jnp.bfloat16),
        scratch_shapes=[
            pltpu.VMEM((H, IC), jnp.bfloat16),
            pltpu.SMEM((1,), jnp.int32),
        ],
    )(x2d, sgu, gs)


def _s2_body(act_ref, sd_ref, g_ref, w_ref, out_ref):
    sh = jnp.dot(act_ref[...], sd_ref[...], preferred_element_type=jnp.float32)
    g0 = g_ref[0, :, 0, :]
    g1 = g_ref[0, :, 1, :]
    w0 = w_ref[0, :, 0:1]
    w1 = w_ref[0, :, 1:2]
    out_ref[...] = w0 * g0 + w1 * g1 + sh


def _s2(act, sdb, g4, wv):
    return pl.pallas_call(
        _s2_body,
        grid=(TBLK,),
        in_specs=[
            pl.BlockSpec((BM, I), lambda t: (t, 0)),
            pl.BlockSpec((I, H), lambda t: (0, 0)),
            pl.BlockSpec((1, BM, K, H), lambda t: (t, 0, 0, 0)),
            pl.BlockSpec((1, BM, K), lambda t: (t, 0, 0)),
        ],
        out_specs=pl.BlockSpec((BM, H), lambda t: (t, 0)),
        out_shape=jax.ShapeDtypeStruct((T, H), jnp.float32),
    )(act, sdb, g4, wv)


def kernel(hidden_states, router_weight, w1, w3, w2, shared_gate_up,
           shared_down):
    b, s, h = hidden_states.shape
    x2d = hidden_states.reshape(T, H)
    rwt = router_weight.T

    idx, wv, cnt = _router(x2d, rwt)
    idx16 = idx.reshape(16, 256)

    pos, meta, posk = _sort_sc(idx16, cnt.reshape(16, 16))

    gs = _s1a(x2d, shared_gate_up)
    act_sh = _s1b(x2d, shared_gate_up, gs)

    xs = _scatter_rows_sc(x2d, posk)
    gsx = _g1a(meta, xs, w1)
    act = _g1b(meta, xs, w3, gsx)
    ys = _g2(meta, act, w2)
    g = _gather_rows_sc(ys, pos.reshape(-1), T * K)
    g4 = g.reshape(TBLK, BM, K, H)

    out = _s2(act_sh, shared_down.astype(jnp.bfloat16), g4, wv)
    return out.reshape(b, s, h)
